# Initial kernel scaffold; baseline (speedup 1.0000x reference)
import jax
import jax.numpy as jnp
from jax.experimental import pallas as pl


def kernel(char_prem_batch, char_hypo_batch, char_prem_masks, char_hypo_masks, table):
    raise NotImplementedError("write your pallas kernel here")


# speedup vs baseline: 11.6383x; 11.6383x over previous
import functools

import jax
import jax.numpy as jnp
from jax import lax
from jax.experimental import pallas as pl
from jax.experimental.pallas import tpu as pltpu
from jax.experimental.pallas import tpu_sc as plsc

S, W, B, V, D = 128, 16, 256, 1000, 64
L = 16
BC = B // L
SEQ_PER_TILE = 4

_mesh = plsc.VectorSubcoreMesh(core_axis_name="c", subcore_axis_name="s")


@functools.partial(
    pl.kernel,
    out_type=(
        jax.ShapeDtypeStruct((S, B, D), jnp.float32),
        jax.ShapeDtypeStruct((S, B, D), jnp.float32),
    ),
    mesh=_mesh,
    compiler_params=pltpu.CompilerParams(needs_layout_passes=False),
    scratch_types=[
        pltpu.VMEM((V * D,), jnp.float32),
        pltpu.VMEM((W, B), jnp.int32),
        pltpu.VMEM((W, B), jnp.float32),
        pltpu.VMEM((B, D), jnp.float32),
    ],
)
def _encode(prem_idx, hypo_idx, prem_mask, hypo_mask, table_t,
            out_p, out_h, tbl_v, idx_v, mask_v, out_v):
    wid = lax.axis_index("s") * 2 + lax.axis_index("c")
    pltpu.sync_copy(table_t, tbl_v)
    lane = lax.iota(jnp.int32, L)

    for idx_hbm, mask_hbm, out_hbm in (
        (prem_idx, prem_mask, out_p),
        (hypo_idx, hypo_mask, out_h),
    ):
        for j in range(SEQ_PER_TILE):
            s = wid * SEQ_PER_TILE + j
            pltpu.sync_copy(idx_hbm.at[s], idx_v)
            pltpu.sync_copy(mask_hbm.at[s], mask_v)

            def bc_body(bc, _):
                iv = [idx_v[w, pl.ds(bc * L, L)] for w in range(W)]
                mv = [mask_v[w, pl.ds(bc * L, L)] * (1.0 / W) for w in range(W)]
                b_idx = lane + bc * L

                def d_body(dd, _):
                    base = dd * V
                    acc = plsc.load_gather(tbl_v, [iv[0] + base]) * mv[0]
                    for w in range(1, W):
                        acc = acc + plsc.load_gather(tbl_v, [iv[w] + base]) * mv[w]
                    plsc.store_scatter(
                        out_v, [b_idx, jnp.full((L,), dd, jnp.int32)], acc)
                    return 0

                lax.fori_loop(0, D, d_body, 0)
                return 0

            lax.fori_loop(0, BC, bc_body, 0)
            pltpu.sync_copy(out_v, out_hbm.at[s])


def kernel(char_prem_batch, char_hypo_batch, char_prem_masks, char_hypo_masks,
           table):
    table_t = table.T.reshape(-1)
    return _encode(char_prem_batch.astype(jnp.int32),
                   char_hypo_batch.astype(jnp.int32),
                   char_prem_masks, char_hypo_masks, table_t)


# speedup vs baseline: 12.8110x; 1.1008x over previous
import functools

import jax
import jax.numpy as jnp
from jax import lax
from jax.experimental import pallas as pl
from jax.experimental.pallas import tpu as pltpu
from jax.experimental.pallas import tpu_sc as plsc

S, W, B, V, D = 128, 16, 256, 1000, 64
L = 16
BC = B // L
SEQ_PER_TILE = 4

_mesh = plsc.VectorSubcoreMesh(core_axis_name="c", subcore_axis_name="s")


@functools.partial(
    pl.kernel,
    out_type=(
        jax.ShapeDtypeStruct((S, B, D), jnp.float32),
        jax.ShapeDtypeStruct((S, B, D), jnp.float32),
    ),
    mesh=_mesh,
    compiler_params=pltpu.CompilerParams(needs_layout_passes=False),
    scratch_types=[
        pltpu.VMEM((V * D,), jnp.float32),
        pltpu.VMEM((W, B), jnp.int32),
        pltpu.VMEM((W, B), jnp.float32),
        pltpu.VMEM((B, D), jnp.float32),
    ],
)
def _encode(prem_idx, hypo_idx, prem_mask, hypo_mask, table_t,
            out_p, out_h, tbl_v, idx_v, mask_v, out_v):
    wid = lax.axis_index("s") * 2 + lax.axis_index("c")
    pltpu.sync_copy(table_t, tbl_v)
    lane = lax.iota(jnp.int32, L)

    for idx_hbm, mask_hbm, out_hbm in (
        (prem_idx, prem_mask, out_p),
        (hypo_idx, hypo_mask, out_h),
    ):
        for j in range(SEQ_PER_TILE):
            s = wid * SEQ_PER_TILE + j
            pltpu.sync_copy(idx_hbm.at[s], idx_v)
            pltpu.sync_copy(mask_hbm.at[s], mask_v)

            def bc_body(bc, _):
                iv = [idx_v[w, pl.ds(bc * L, L)] for w in range(W)]
                mv = [mask_v[w, pl.ds(bc * L, L)] * (1.0 / W) for w in range(W)]
                b_idx = lane + bc * L

                def d_body(d4, _):
                    for k in range(4):
                        dd = d4 * 4 + k
                        base = dd * V
                        terms = [plsc.load_gather(tbl_v, [iv[w] + base]) * mv[w]
                                 for w in range(W)]
                        while len(terms) > 1:
                            terms = [terms[i] + terms[i + 1]
                                     for i in range(0, len(terms), 2)]
                        plsc.store_scatter(
                            out_v, [b_idx, jnp.full((L,), dd, jnp.int32)],
                            terms[0])
                    return 0

                lax.fori_loop(0, D // 4, d_body, 0)
                return 0

            lax.fori_loop(0, BC, bc_body, 0)
            pltpu.sync_copy(out_v, out_hbm.at[s])


def kernel(char_prem_batch, char_hypo_batch, char_prem_masks, char_hypo_masks,
           table):
    table_t = table.T.reshape(-1)
    return _encode(char_prem_batch.astype(jnp.int32),
                   char_hypo_batch.astype(jnp.int32),
                   char_prem_masks, char_hypo_masks, table_t)


# speedup vs baseline: 14.4254x; 1.1260x over previous
import functools

import jax
import jax.numpy as jnp
from jax import lax
from jax.experimental import pallas as pl
from jax.experimental.pallas import tpu as pltpu
from jax.experimental.pallas import tpu_sc as plsc

S, W, B, V, D = 128, 16, 256, 1000, 64
L = 16
BC = B // L
SEQ_PER_TILE = 4
NC = D // L

_mesh = plsc.VectorSubcoreMesh(core_axis_name="c", subcore_axis_name="s")
_IN_BOUNDS = lax.GatherScatterMode.PROMISE_IN_BOUNDS


_DNUMS = lax.GatherDimensionNumbers(
    offset_dims=(), collapsed_slice_dims=(0,), start_index_map=(0,))


def _bcast_lane(vec, lidx):
    return lax.gather(vec, lidx[:, None], _DNUMS, (1,), mode=_IN_BOUNDS)


@functools.partial(
    pl.kernel,
    out_type=(
        jax.ShapeDtypeStruct((S, B, D), jnp.float32),
        jax.ShapeDtypeStruct((S, B, D), jnp.float32),
    ),
    mesh=_mesh,
    compiler_params=pltpu.CompilerParams(needs_layout_passes=False),
    scratch_types=[
        pltpu.VMEM((V * D,), jnp.float32),
        pltpu.VMEM((W, B), jnp.int32),
        pltpu.VMEM((W, B), jnp.float32),
        pltpu.VMEM((B, D), jnp.float32),
    ],
)
def _encode(prem_idx, hypo_idx, prem_mask, hypo_mask, table_f,
            out_p, out_h, tbl_v, idx_v, mask_v, out_v):
    wid = lax.axis_index("s") * 2 + lax.axis_index("c")
    pltpu.sync_copy(table_f, tbl_v)
    iota = lax.iota(jnp.int32, L)

    for idx_hbm, mask_hbm, out_hbm in (
        (prem_idx, prem_mask, out_p),
        (hypo_idx, hypo_mask, out_h),
    ):
        for j in range(SEQ_PER_TILE):
            s = wid * SEQ_PER_TILE + j
            pltpu.sync_copy(idx_hbm.at[s], idx_v)
            pltpu.sync_copy(mask_hbm.at[s], mask_v)

            def bc_body(bc, _):
                iv = [idx_v[w, pl.ds(bc * L, L)] for w in range(W)]
                mv = [mask_v[w, pl.ds(bc * L, L)] * (1.0 / W) for w in range(W)]

                def l_body(l, _):
                    lidx = jnp.full((L,), l, jnp.int32)
                    tb = [_bcast_lane(iv[w], lidx) * D + iota for w in range(W)]
                    mb = [_bcast_lane(mv[w], lidx) for w in range(W)]
                    b_abs = bc * L + l
                    for c in range(NC):
                        off = c * L
                        terms = [plsc.load_gather(tbl_v, [tb[w] + off]) * mb[w]
                                 for w in range(W)]
                        while len(terms) > 1:
                            terms = [terms[i] + terms[i + 1]
                                     for i in range(0, len(terms), 2)]
                        out_v[b_abs, pl.ds(off, L)] = terms[0]
                    return 0

                lax.fori_loop(0, L, l_body, 0)
                return 0

            lax.fori_loop(0, BC, bc_body, 0)
            pltpu.sync_copy(out_v, out_hbm.at[s])


def kernel(char_prem_batch, char_hypo_batch, char_prem_masks, char_hypo_masks,
           table):
    return _encode(char_prem_batch.astype(jnp.int32),
                   char_hypo_batch.astype(jnp.int32),
                   char_prem_masks, char_hypo_masks, table.reshape(-1))
